# baseline (speedup 1.0000x reference)
import functools

import jax
import jax.numpy as jnp
from jax import lax
from jax.experimental import pallas as pl
from jax.experimental.pallas import tpu as pltpu
from jax.experimental.pallas import tpu_sc as plsc

_N = 10000
_E = 320000
_D = 128
_H = 8
_DH = 16

_NTILES = 32
_EPT = _E // _NTILES
_CH = 80
_NCH = _EPT // _CH
_GR = _CH // 16
_NPAD = 10240
_NPT = _NPAD // 16
_AW = 144


def _ln(x, g, b):
    mu = jnp.mean(x, axis=-1, keepdims=True)
    var = jnp.mean((x - mu) * (x - mu), axis=-1, keepdims=True)
    return (x - mu) / jnp.sqrt(var + 1e-5) * g + b


def _qkv_body(x_ref, wq_ref, bq_ref, wk_ref, bk_ref, wv_ref, bv_ref,
              g1_ref, b1_ref, q_ref, kv_ref):
    hn = _ln(x_ref[...], g1_ref[...], b1_ref[...])
    dn = (((1,), (1,)), ((), ()))
    q = lax.dot_general(hn, wq_ref[...], dn,
                        preferred_element_type=jnp.float32) + bq_ref[...]
    k = lax.dot_general(hn, wk_ref[...], dn,
                        preferred_element_type=jnp.float32) + bk_ref[...]
    v = lax.dot_general(hn, wv_ref[...], dn,
                        preferred_element_type=jnp.float32) + bv_ref[...]
    q_ref[...] = q * 0.25
    kv_ref[...] = jnp.concatenate([k, v], axis=1)


def _qkv(x, Wq, bq, Wk, bk, Wv, bv, g1, b1):
    blk = 400
    grid = _N // blk
    full = lambda s: pl.BlockSpec(s, lambda i: (0, 0))
    return pl.pallas_call(
        _qkv_body,
        grid=(grid,),
        in_specs=[
            pl.BlockSpec((blk, _D), lambda i: (i, 0)),
            full((_D, _D)), full((1, _D)),
            full((_D, _D)), full((1, _D)),
            full((_D, _D)), full((1, _D)),
            full((1, _D)), full((1, _D)),
        ],
        out_specs=[
            pl.BlockSpec((blk, _D), lambda i: (i, 0)),
            pl.BlockSpec((blk, 2 * _D), lambda i: (i, 0)),
        ],
        out_shape=[
            jax.ShapeDtypeStruct((_N, _D), jnp.float32),
            jax.ShapeDtypeStruct((_N, 2 * _D), jnp.float32),
        ],
    )(x, Wq, bq.reshape(1, _D), Wk, bk.reshape(1, _D),
      Wv, bv.reshape(1, _D), g1.reshape(1, _D), b1.reshape(1, _D))


def _ebias_body(ea_ref, we_ref, be_ref, out_ref):
    dn = (((1,), (1,)), ((), ()))
    eb = lax.dot_general(ea_ref[...], we_ref[...], dn,
                         preferred_element_type=jnp.float32) + be_ref[...]
    out_ref[...] = jnp.concatenate(
        [eb, jnp.zeros((eb.shape[0], _H), jnp.float32)], axis=1)


def _ebias(edge_attr, We, be):
    blk = 2000
    grid = _E // blk
    return pl.pallas_call(
        _ebias_body,
        grid=(grid,),
        in_specs=[
            pl.BlockSpec((blk, _D), lambda i: (i, 0)),
            pl.BlockSpec((_H, _D), lambda i: (0, 0)),
            pl.BlockSpec((1, _H), lambda i: (0, 0)),
        ],
        out_specs=pl.BlockSpec((blk, 2 * _H), lambda i: (i, 0)),
        out_shape=jax.ShapeDtypeStruct((_E, 2 * _H), jnp.float32),
    )(edge_attr, We, be.reshape(1, _H))


def _edge_body(q_hbm, kv_hbm, src_hbm, dst_hbm, eb_hbm,
               acc_hbm, den_hbm,
               kvbuf, qbuf, msgbuf, srcbuf, dstbuf, ebuf, exbuf, aug, den_sp):
    c = lax.axis_index("c")
    s = lax.axis_index("s")
    wid = c * 16 + s
    ebase = wid * _EPT
    nbase = s * _NPT
    zero16 = jnp.zeros((16,), jnp.float32)
    lane = lax.iota(jnp.int32, 16)

    def _zrow(i, carry):
        for j in range(_D // 16):
            msgbuf[i, pl.ds(j * 16, 16)] = zero16
        return carry
    lax.fori_loop(0, _CH, _zrow, 0)
    for j in range(_NPT // _CH):
        pltpu.sync_copy(msgbuf, aug.at[pl.ds(nbase + j * _CH, _CH)])
    pltpu.sync_copy(msgbuf.at[pl.ds(0, _NPAD // 16 // 16)],
                    den_sp.at[pl.ds(s * (_NPAD // 16 // 16), _NPAD // 16 // 16)])
    plsc.subcore_barrier()

    def _chunk(ci, carry):
        base = ebase + ci * _CH
        pltpu.sync_copy(src_hbm.at[pl.ds(base, _CH)], srcbuf)
        pltpu.sync_copy(dst_hbm.at[pl.ds(base, _CH)], dstbuf)
        pltpu.sync_copy(eb_hbm.at[pl.ds(base * 16, _CH * 16)], ebuf)
        pltpu.sync_copy(kv_hbm.at[srcbuf], kvbuf)
        pltpu.sync_copy(q_hbm.at[dstbuf], qbuf)

        @plsc.parallel_loop(0, _GR, 1)
        def _gsrc(g):
            dv = dstbuf[pl.ds(g * 16, 16)]
            srcbuf[pl.ds(g * 16, 16)] = lax.shift_right_logical(dv, 4)

        @plsc.parallel_loop(0, _GR * _H, 1)
        def _ga(gh):
            g = lax.shift_right_logical(gh, 3)
            h = gh & 7
            e_idx = g * 16 + lane
            hbase = h * _DH
            acc_v = plsc.load_gather(ebuf, [e_idx * 16 + h])
            acc_b = jnp.zeros((16,), jnp.float32)
            for d in range(0, _DH, 2):
                col = jnp.full((16,), hbase + d, jnp.int32)
                colb = jnp.full((16,), hbase + d + 1, jnp.int32)
                acc_v = acc_v + (plsc.load_gather(kvbuf, [e_idx, col])
                                 * plsc.load_gather(qbuf, [e_idx, col]))
                acc_b = acc_b + (plsc.load_gather(kvbuf, [e_idx, colb])
                                 * plsc.load_gather(qbuf, [e_idx, colb]))
            ex_v = jnp.exp(acc_v + acc_b)
            for d in range(_DH):
                colv = jnp.full((16,), _D + hbase + d, jnp.int32)
                colm = jnp.full((16,), hbase + d, jnp.int32)
                vv = plsc.load_gather(kvbuf, [e_idx, colv])
                plsc.store_scatter(msgbuf, [e_idx, colm], vv * ex_v)
            plsc.store_scatter(exbuf, [e_idx * _H + h], ex_v)
        pltpu.sync_copy(msgbuf, aug.at[dstbuf], add=True)

        @plsc.parallel_loop(0, _CH, 1)
        def _zmsg(i):
            for j in range(_D // 16):
                msgbuf[i, pl.ds(j * 16, 16)] = zero16

        @plsc.parallel_loop(0, _GR, 1)
        def _gc(g):
            e_idx = g * 16 + lane
            dv = dstbuf[pl.ds(g * 16, 16)]
            colbase = ((lax.shift_right_logical(dv, 1) & 7) * 16
                       + (dv & 1) * 8)
            for h in range(_H):
                exv = plsc.load_gather(exbuf, [e_idx * _H + h])
                plsc.store_scatter(msgbuf, [e_idx, colbase + h], exv)
        pltpu.sync_copy(msgbuf, den_sp.at[srcbuf], add=True)
        return carry
    lax.fori_loop(0, _NCH, _chunk, 0)

    plsc.subcore_barrier()
    pltpu.sync_copy(aug.at[pl.ds(nbase, _NPT)],
                    acc_hbm.at[c, pl.ds(nbase, _NPT)])
    nd = _NPAD // 16 // 16
    pltpu.sync_copy(den_sp.at[pl.ds(s * nd, nd)],
                    den_hbm.at[c, pl.ds(s * nd, nd)])


def _edge_pass(q, kv, src, dst, ebias):
    mesh = plsc.VectorSubcoreMesh(core_axis_name="c", subcore_axis_name="s")
    f = pl.kernel(
        _edge_body,
        out_type=[
            jax.ShapeDtypeStruct((2, _NPAD, _D), jnp.float32),
            jax.ShapeDtypeStruct((2, _NPAD // 16, _D), jnp.float32),
        ],
        mesh=mesh,
        scratch_types=[
            pltpu.VMEM((_CH, 2 * _D), jnp.float32),
            pltpu.VMEM((_CH, _D), jnp.float32),
            pltpu.VMEM((_CH, _D), jnp.float32),
            pltpu.VMEM((_CH,), jnp.int32),
            pltpu.VMEM((_CH,), jnp.int32),
            pltpu.VMEM((_CH * 16,), jnp.float32),
            pltpu.VMEM((_CH * _H,), jnp.float32),
            pltpu.VMEM_SHARED((_NPAD, _D), jnp.float32),
            pltpu.VMEM_SHARED((_NPAD // 16, _D), jnp.float32),
        ],
        compiler_params=pltpu.CompilerParams(needs_layout_passes=False),
    )
    return f(q, kv, src, dst, ebias)


def _comb_body(acc_ref, den_ref, x_ref, wo_ref, bo_ref, g2_ref, b2_ref,
               w1_ref, bf1_ref, w2_ref, bf2_ref, y_ref):
    acc = acc_ref[0] + acc_ref[1]
    den = den_ref[0] + den_ref[1]
    li = lax.broadcasted_iota(jnp.int32, (_H, _D), 1) // _DH
    hi = lax.broadcasted_iota(jnp.int32, (_H, _D), 0)
    rep = (li == hi).astype(jnp.float32)
    dn = (((1,), (0,)), ((), ()))
    den_r = lax.dot_general(den, rep, dn, preferred_element_type=jnp.float32)
    out = acc / (den_r + 1e-16)
    dt = (((1,), (1,)), ((), ()))
    attn = lax.dot_general(out, wo_ref[...], dt,
                           preferred_element_type=jnp.float32) + bo_ref[...]
    h = x_ref[...] + attn
    hn2 = _ln(h, g2_ref[...], b2_ref[...])
    f1 = lax.dot_general(hn2, w1_ref[...], dt,
                         preferred_element_type=jnp.float32) + bf1_ref[...]
    f1 = jnp.maximum(f1, 0.0)
    ffn = lax.dot_general(f1, w2_ref[...], dt,
                          preferred_element_type=jnp.float32) + bf2_ref[...]
    y_ref[...] = h + ffn


def _combine(acc, den, x, Wo, bo, g2, b2, W1, bf1, W2, bf2):
    blk = 400
    grid = _N // blk
    full = lambda s: pl.BlockSpec(s, lambda i: tuple(0 for _ in s))
    return pl.pallas_call(
        _comb_body,
        grid=(grid,),
        in_specs=[
            pl.BlockSpec((2, blk, _D), lambda i: (0, i, 0)),
            pl.BlockSpec((2, blk, _H), lambda i: (0, i, 0)),
            pl.BlockSpec((blk, _D), lambda i: (i, 0)),
            full((_D, _D)), full((1, _D)),
            full((1, _D)), full((1, _D)),
            full((2 * _D, _D)), full((1, 2 * _D)),
            full((_D, 2 * _D)), full((1, _D)),
        ],
        out_specs=pl.BlockSpec((blk, _D), lambda i: (i, 0)),
        out_shape=jax.ShapeDtypeStruct((_N, _D), jnp.float32),
    )(acc, den, x, Wo, bo.reshape(1, _D), g2.reshape(1, _D), b2.reshape(1, _D),
      W1, bf1.reshape(1, 2 * _D), W2, bf2.reshape(1, _D))


def kernel(x, edge_index, edge_attr, Wq, bq, Wk, bk, Wv, bv, We, be,
           Wo, bo, g1, b1, g2, b2, W1, bf1, W2, bf2):
    src = edge_index[0]
    dst = edge_index[1]
    q, kv = _qkv(x, Wq, bq, Wk, bk, Wv, bv, g1, b1)
    ebias = _ebias(edge_attr, We, be).reshape(-1)
    acc, den = _edge_pass(q, kv, src, dst, ebias)
    den = den.reshape(2, _NPAD, _H)
    return _combine(acc, den, x, Wo, bo, g2, b2, W1, bf1, W2, bf2)


# speedup vs baseline: 1.0080x; 1.0080x over previous
import functools

import jax
import jax.numpy as jnp
from jax import lax
from jax.experimental import pallas as pl
from jax.experimental.pallas import tpu as pltpu
from jax.experimental.pallas import tpu_sc as plsc

_N = 10000
_E = 320000
_D = 128
_H = 8
_DH = 16

_NTILES = 32
_EPT = _E // _NTILES
_CH = 80
_NCH = _EPT // _CH
_GR = _CH // 16
_NPAD = 10240
_NPT = _NPAD // 16
_AW = 144


def _ln(x, g, b):
    mu = jnp.mean(x, axis=-1, keepdims=True)
    var = jnp.mean((x - mu) * (x - mu), axis=-1, keepdims=True)
    return (x - mu) / jnp.sqrt(var + 1e-5) * g + b


def _qkv_body(x_ref, wq_ref, bq_ref, wk_ref, bk_ref, wv_ref, bv_ref,
              g1_ref, b1_ref, q_ref, kv_ref):
    hn = _ln(x_ref[...], g1_ref[...], b1_ref[...])
    dn = (((1,), (1,)), ((), ()))
    q = lax.dot_general(hn, wq_ref[...], dn,
                        preferred_element_type=jnp.float32) + bq_ref[...]
    k = lax.dot_general(hn, wk_ref[...], dn,
                        preferred_element_type=jnp.float32) + bk_ref[...]
    v = lax.dot_general(hn, wv_ref[...], dn,
                        preferred_element_type=jnp.float32) + bv_ref[...]
    q_ref[...] = q * 0.25
    kv_ref[...] = jnp.concatenate([k, v], axis=1)


def _qkv(x, Wq, bq, Wk, bk, Wv, bv, g1, b1):
    blk = 400
    grid = _N // blk
    full = lambda s: pl.BlockSpec(s, lambda i: (0, 0))
    return pl.pallas_call(
        _qkv_body,
        grid=(grid,),
        in_specs=[
            pl.BlockSpec((blk, _D), lambda i: (i, 0)),
            full((_D, _D)), full((1, _D)),
            full((_D, _D)), full((1, _D)),
            full((_D, _D)), full((1, _D)),
            full((1, _D)), full((1, _D)),
        ],
        out_specs=[
            pl.BlockSpec((blk, _D), lambda i: (i, 0)),
            pl.BlockSpec((blk, 2 * _D), lambda i: (i, 0)),
        ],
        out_shape=[
            jax.ShapeDtypeStruct((_N, _D), jnp.float32),
            jax.ShapeDtypeStruct((_N, 2 * _D), jnp.float32),
        ],
    )(x, Wq, bq.reshape(1, _D), Wk, bk.reshape(1, _D),
      Wv, bv.reshape(1, _D), g1.reshape(1, _D), b1.reshape(1, _D))


def _ebias_body(ea_ref, we_ref, be_ref, src_ref, dst_ref, out_ref):
    dn = (((1,), (1,)), ((), ()))
    eb = lax.dot_general(ea_ref[...], we_ref[...], dn,
                         preferred_element_type=jnp.float32) + be_ref[...]
    srcf = lax.bitcast_convert_type(src_ref[...], jnp.float32)
    dstf = lax.bitcast_convert_type(dst_ref[...], jnp.float32)
    blk = eb.shape[0]
    out_ref[...] = jnp.concatenate(
        [eb, srcf, dstf, jnp.zeros((blk, 6), jnp.float32)], axis=1)


def _ebias(edge_attr, We, be, src, dst):
    blk = 2000
    grid = _E // blk
    return pl.pallas_call(
        _ebias_body,
        grid=(grid,),
        in_specs=[
            pl.BlockSpec((blk, _D), lambda i: (i, 0)),
            pl.BlockSpec((_H, _D), lambda i: (0, 0)),
            pl.BlockSpec((1, _H), lambda i: (0, 0)),
            pl.BlockSpec((blk, 1), lambda i: (i, 0)),
            pl.BlockSpec((blk, 1), lambda i: (i, 0)),
        ],
        out_specs=pl.BlockSpec((blk, 2 * _H), lambda i: (i, 0)),
        out_shape=jax.ShapeDtypeStruct((_E, 2 * _H), jnp.float32),
    )(edge_attr, We, be.reshape(1, _H), src.reshape(_E, 1), dst.reshape(_E, 1))


def _edge_body(q_hbm, kv_hbm, eb_hbm,
               acc_hbm, den_hbm,
               kvbuf, qbuf, msgbuf, srcbuf, dstbuf, ebuf, exbuf, aug, den_sp,
               sem1, sem2):
    c = lax.axis_index("c")
    s = lax.axis_index("s")
    wid = c * 16 + s
    ebase = wid * _EPT
    nbase = s * _NPT
    zero16 = jnp.zeros((16,), jnp.float32)
    lane = lax.iota(jnp.int32, 16)

    def _zrow(i, carry):
        for j in range(_D // 16):
            msgbuf[i, pl.ds(j * 16, 16)] = zero16
        return carry
    lax.fori_loop(0, _CH, _zrow, 0)
    for j in range(_NPT // _CH):
        pltpu.sync_copy(msgbuf, aug.at[pl.ds(nbase + j * _CH, _CH)])
    pltpu.sync_copy(msgbuf.at[pl.ds(0, _NPAD // 16 // 16)],
                    den_sp.at[pl.ds(s * (_NPAD // 16 // 16), _NPAD // 16 // 16)])
    plsc.subcore_barrier()

    def _chunk(ci, carry):
        base = ebase + ci * _CH
        pltpu.sync_copy(eb_hbm.at[pl.ds(base * 16, _CH * 16)], ebuf)

        @plsc.parallel_loop(0, _GR, 1)
        def _unpack(g):
            e_idx = g * 16 + lane
            sv = plsc.bitcast(plsc.load_gather(ebuf, [e_idx * 16 + _H]),
                              jnp.int32)
            dv = plsc.bitcast(plsc.load_gather(ebuf, [e_idx * 16 + _H + 1]),
                              jnp.int32)
            srcbuf[pl.ds(g * 16, 16)] = sv
            dstbuf[pl.ds(g * 16, 16)] = dv

        kvd = pltpu.async_copy(kv_hbm.at[srcbuf], kvbuf, sem1)
        qd = pltpu.async_copy(q_hbm.at[dstbuf], qbuf, sem2)
        kvd.wait()
        qd.wait()

        @plsc.parallel_loop(0, _GR, 1)
        def _gsrc(g):
            dv = dstbuf[pl.ds(g * 16, 16)]
            srcbuf[pl.ds(g * 16, 16)] = lax.shift_right_logical(dv, 4)

        @plsc.parallel_loop(0, _GR * _H, 1)
        def _ga(gh):
            g = lax.shift_right_logical(gh, 3)
            h = gh & 7
            e_idx = g * 16 + lane
            hbase = h * _DH
            acc_v = plsc.load_gather(ebuf, [e_idx * 16 + h])
            for d in range(_DH):
                col = jnp.full((16,), hbase + d, jnp.int32)
                kvv = plsc.load_gather(kvbuf, [e_idx, col])
                qv = plsc.load_gather(qbuf, [e_idx, col])
                acc_v = acc_v + kvv * qv
            ex_v = jnp.exp(acc_v)
            for d in range(_DH):
                colv = jnp.full((16,), _D + hbase + d, jnp.int32)
                colm = jnp.full((16,), hbase + d, jnp.int32)
                vv = plsc.load_gather(kvbuf, [e_idx, colv])
                plsc.store_scatter(msgbuf, [e_idx, colm], vv * ex_v)
            plsc.store_scatter(exbuf, [e_idx * _H + h], ex_v)
        pltpu.sync_copy(msgbuf, aug.at[dstbuf], add=True)

        @plsc.parallel_loop(0, _CH, 1)
        def _zmsg(i):
            for j in range(_D // 16):
                msgbuf[i, pl.ds(j * 16, 16)] = zero16

        @plsc.parallel_loop(0, _GR, 1)
        def _gc(g):
            e_idx = g * 16 + lane
            dv = dstbuf[pl.ds(g * 16, 16)]
            colbase = ((lax.shift_right_logical(dv, 1) & 7) * 16
                       + (dv & 1) * 8)
            for h in range(_H):
                exv = plsc.load_gather(exbuf, [e_idx * _H + h])
                plsc.store_scatter(msgbuf, [e_idx, colbase + h], exv)
        pltpu.sync_copy(msgbuf, den_sp.at[srcbuf], add=True)
        return carry
    lax.fori_loop(0, _NCH, _chunk, 0)

    plsc.subcore_barrier()
    pltpu.sync_copy(aug.at[pl.ds(nbase, _NPT)],
                    acc_hbm.at[c, pl.ds(nbase, _NPT)])
    nd = _NPAD // 16 // 16
    pltpu.sync_copy(den_sp.at[pl.ds(s * nd, nd)],
                    den_hbm.at[c, pl.ds(s * nd, nd)])


def _edge_pass(q, kv, ebias):
    mesh = plsc.VectorSubcoreMesh(core_axis_name="c", subcore_axis_name="s")
    f = pl.kernel(
        _edge_body,
        out_type=[
            jax.ShapeDtypeStruct((2, _NPAD, _D), jnp.float32),
            jax.ShapeDtypeStruct((2, _NPAD // 16, _D), jnp.float32),
        ],
        mesh=mesh,
        scratch_types=[
            pltpu.VMEM((_CH, 2 * _D), jnp.float32),
            pltpu.VMEM((_CH, _D), jnp.float32),
            pltpu.VMEM((_CH, _D), jnp.float32),
            pltpu.VMEM((_CH,), jnp.int32),
            pltpu.VMEM((_CH,), jnp.int32),
            pltpu.VMEM((_CH * 16,), jnp.float32),
            pltpu.VMEM((_CH * _H,), jnp.float32),
            pltpu.VMEM_SHARED((_NPAD, _D), jnp.float32),
            pltpu.VMEM_SHARED((_NPAD // 16, _D), jnp.float32),
            pltpu.SemaphoreType.DMA,
            pltpu.SemaphoreType.DMA,
        ],
        compiler_params=pltpu.CompilerParams(needs_layout_passes=False),
    )
    return f(q, kv, ebias)


def _comb_body(acc_ref, den_ref, x_ref, wo_ref, bo_ref, g2_ref, b2_ref,
               w1_ref, bf1_ref, w2_ref, bf2_ref, y_ref):
    acc = acc_ref[0] + acc_ref[1]
    den = den_ref[0] + den_ref[1]
    li = lax.broadcasted_iota(jnp.int32, (_H, _D), 1) // _DH
    hi = lax.broadcasted_iota(jnp.int32, (_H, _D), 0)
    rep = (li == hi).astype(jnp.float32)
    dn = (((1,), (0,)), ((), ()))
    den_r = lax.dot_general(den, rep, dn, preferred_element_type=jnp.float32)
    out = acc / (den_r + 1e-16)
    dt = (((1,), (1,)), ((), ()))
    attn = lax.dot_general(out, wo_ref[...], dt,
                           preferred_element_type=jnp.float32) + bo_ref[...]
    h = x_ref[...] + attn
    hn2 = _ln(h, g2_ref[...], b2_ref[...])
    f1 = lax.dot_general(hn2, w1_ref[...], dt,
                         preferred_element_type=jnp.float32) + bf1_ref[...]
    f1 = jnp.maximum(f1, 0.0)
    ffn = lax.dot_general(f1, w2_ref[...], dt,
                          preferred_element_type=jnp.float32) + bf2_ref[...]
    y_ref[...] = h + ffn


def _combine(acc, den, x, Wo, bo, g2, b2, W1, bf1, W2, bf2):
    blk = 400
    grid = _N // blk
    full = lambda s: pl.BlockSpec(s, lambda i: tuple(0 for _ in s))
    return pl.pallas_call(
        _comb_body,
        grid=(grid,),
        in_specs=[
            pl.BlockSpec((2, blk, _D), lambda i: (0, i, 0)),
            pl.BlockSpec((2, blk, _H), lambda i: (0, i, 0)),
            pl.BlockSpec((blk, _D), lambda i: (i, 0)),
            full((_D, _D)), full((1, _D)),
            full((1, _D)), full((1, _D)),
            full((2 * _D, _D)), full((1, 2 * _D)),
            full((_D, 2 * _D)), full((1, _D)),
        ],
        out_specs=pl.BlockSpec((blk, _D), lambda i: (i, 0)),
        out_shape=jax.ShapeDtypeStruct((_N, _D), jnp.float32),
    )(acc, den, x, Wo, bo.reshape(1, _D), g2.reshape(1, _D), b2.reshape(1, _D),
      W1, bf1.reshape(1, 2 * _D), W2, bf2.reshape(1, _D))


def kernel(x, edge_index, edge_attr, Wq, bq, Wk, bk, Wv, bv, We, be,
           Wo, bo, g1, b1, g2, b2, W1, bf1, W2, bf2):
    src = edge_index[0]
    dst = edge_index[1]
    q, kv = _qkv(x, Wq, bq, Wk, bk, Wv, bv, g1, b1)
    ebias = _ebias(edge_attr, We, be, src, dst).reshape(-1)
    acc, den = _edge_pass(q, kv, ebias)
    den = den.reshape(2, _NPAD, _H)
    return _combine(acc, den, x, Wo, bo, g2, b2, W1, bf1, W2, bf2)


# speedup vs baseline: 2.4899x; 2.4701x over previous
import functools

import jax
import jax.numpy as jnp
from jax import lax
from jax.experimental import pallas as pl
from jax.experimental.pallas import tpu as pltpu
from jax.experimental.pallas import tpu_sc as plsc

_GDN = lax.GatherDimensionNumbers(
    offset_dims=(), collapsed_slice_dims=(0,), start_index_map=(0,))


def _vgather(v, idx):
    return lax.gather(v, idx[:, None], _GDN, (1,),
                      mode=lax.GatherScatterMode.PROMISE_IN_BOUNDS)


_N = 10000
_E = 320000
_D = 128
_H = 8
_DH = 16

_NTILES = 32
_EPT = _E // _NTILES
_CH = 80
_NCH = _EPT // _CH
_GR = _CH // 16
_NPAD = 10240
_NPT = _NPAD // 16
_AW = 144


def _ln(x, g, b):
    mu = jnp.mean(x, axis=-1, keepdims=True)
    var = jnp.mean((x - mu) * (x - mu), axis=-1, keepdims=True)
    return (x - mu) / jnp.sqrt(var + 1e-5) * g + b


def _qkv_body(x_ref, wq_ref, bq_ref, wk_ref, bk_ref, wv_ref, bv_ref,
              g1_ref, b1_ref, q_ref, kv_ref):
    hn = _ln(x_ref[...], g1_ref[...], b1_ref[...])
    dn = (((1,), (1,)), ((), ()))
    q = lax.dot_general(hn, wq_ref[...], dn,
                        preferred_element_type=jnp.float32) + bq_ref[...]
    k = lax.dot_general(hn, wk_ref[...], dn,
                        preferred_element_type=jnp.float32) + bk_ref[...]
    v = lax.dot_general(hn, wv_ref[...], dn,
                        preferred_element_type=jnp.float32) + bv_ref[...]
    q_ref[...] = q * 0.25
    kv_ref[...] = jnp.concatenate([k, v], axis=1)


def _qkv(x, Wq, bq, Wk, bk, Wv, bv, g1, b1):
    blk = 400
    grid = _N // blk
    full = lambda s: pl.BlockSpec(s, lambda i: (0, 0))
    return pl.pallas_call(
        _qkv_body,
        grid=(grid,),
        in_specs=[
            pl.BlockSpec((blk, _D), lambda i: (i, 0)),
            full((_D, _D)), full((1, _D)),
            full((_D, _D)), full((1, _D)),
            full((_D, _D)), full((1, _D)),
            full((1, _D)), full((1, _D)),
        ],
        out_specs=[
            pl.BlockSpec((blk, _D), lambda i: (i, 0)),
            pl.BlockSpec((blk, 2 * _D), lambda i: (i, 0)),
        ],
        out_shape=[
            jax.ShapeDtypeStruct((_N, _D), jnp.float32),
            jax.ShapeDtypeStruct((_N, 2 * _D), jnp.float32),
        ],
    )(x, Wq, bq.reshape(1, _D), Wk, bk.reshape(1, _D),
      Wv, bv.reshape(1, _D), g1.reshape(1, _D), b1.reshape(1, _D))


def _ebias_body(ea_ref, we_ref, be_ref, src_ref, dst_ref, out_ref):
    dn = (((1,), (1,)), ((), ()))
    eb = lax.dot_general(ea_ref[...], we_ref[...], dn,
                         preferred_element_type=jnp.float32) + be_ref[...]
    srcf = lax.bitcast_convert_type(src_ref[...], jnp.float32)
    dstf = lax.bitcast_convert_type(dst_ref[...], jnp.float32)
    blk = eb.shape[0]
    out_ref[...] = jnp.concatenate(
        [eb, srcf, dstf, jnp.zeros((blk, 6), jnp.float32)], axis=1)


def _ebias(edge_attr, We, be, src, dst):
    blk = 2000
    grid = _E // blk
    return pl.pallas_call(
        _ebias_body,
        grid=(grid,),
        in_specs=[
            pl.BlockSpec((blk, _D), lambda i: (i, 0)),
            pl.BlockSpec((_H, _D), lambda i: (0, 0)),
            pl.BlockSpec((1, _H), lambda i: (0, 0)),
            pl.BlockSpec((blk, 1), lambda i: (i, 0)),
            pl.BlockSpec((blk, 1), lambda i: (i, 0)),
        ],
        out_specs=pl.BlockSpec((blk, 2 * _H), lambda i: (i, 0)),
        out_shape=jax.ShapeDtypeStruct((_E, 2 * _H), jnp.float32),
    )(edge_attr, We, be.reshape(1, _H), src.reshape(_E, 1), dst.reshape(_E, 1))


def _edge_body(q_hbm, kv_hbm, eb_hbm,
               acc_hbm, den_hbm,
               kvbuf, qbuf, msgbuf, srcbuf, dstbuf, ebuf, exbuf, aug, den_sp,
               sem1, sem2):
    c = lax.axis_index("c")
    s = lax.axis_index("s")
    wid = c * 16 + s
    ebase = wid * _EPT
    nbase = s * _NPT
    zero16 = jnp.zeros((16,), jnp.float32)
    lane = lax.iota(jnp.int32, 16)

    def _zrow(i, carry):
        for j in range(_D // 16):
            msgbuf[i, pl.ds(j * 16, 16)] = zero16
        return carry
    lax.fori_loop(0, _CH, _zrow, 0)
    for j in range(_NPT // _CH):
        pltpu.sync_copy(msgbuf, aug.at[pl.ds(nbase + j * _CH, _CH)])
    pltpu.sync_copy(msgbuf.at[pl.ds(0, _NPAD // 16 // 16)],
                    den_sp.at[pl.ds(s * (_NPAD // 16 // 16), _NPAD // 16 // 16)])
    plsc.subcore_barrier()

    def _chunk(ci, carry):
        base = ebase + ci * _CH
        pltpu.sync_copy(eb_hbm.at[pl.ds(base * 16, _CH * 16)], ebuf)

        @plsc.parallel_loop(0, _GR, 1)
        def _unpack(g):
            e_idx = g * 16 + lane
            sv = plsc.bitcast(plsc.load_gather(ebuf, [e_idx * 16 + _H]),
                              jnp.int32)
            dv = plsc.bitcast(plsc.load_gather(ebuf, [e_idx * 16 + _H + 1]),
                              jnp.int32)
            srcbuf[pl.ds(g * 16, 16)] = sv
            dstbuf[pl.ds(g * 16, 16)] = dv

        kvd = pltpu.async_copy(kv_hbm.at[srcbuf], kvbuf, sem1)
        qd = pltpu.async_copy(q_hbm.at[dstbuf], qbuf, sem2)
        kvd.wait()
        qd.wait()

        @plsc.parallel_loop(0, _GR, 1)
        def _gsrc(g):
            dv = dstbuf[pl.ds(g * 16, 16)]
            srcbuf[pl.ds(g * 16, 16)] = lax.shift_right_logical(dv, 4)

        @plsc.parallel_loop(0, _CH, 1)
        def _ga(i):
            eb_row = ebuf[pl.ds(i * 16, 16)]
            sums = jnp.zeros((16,), jnp.float32)
            for h in range(_H):
                kvh = kvbuf[i, pl.ds(h * _DH, _DH)]
                qh = qbuf[i, pl.ds(h * _DH, _DH)]
                cs = plsc.cumsum(kvh * qh)
                sums = sums + jnp.where(lane == h,
                                        jnp.full((16,), cs[15], jnp.float32),
                                        0.0)
            exv = jnp.exp(sums + eb_row)
            for h in range(_H):
                exb = jnp.full((16,), exv[h], jnp.float32)
                vh = kvbuf[i, pl.ds(_D + h * _DH, _DH)]
                msgbuf[i, pl.ds(h * _DH, _DH)] = vh * exb
            exbuf[pl.ds(i * 16, 16)] = exv
        pltpu.sync_copy(msgbuf, aug.at[dstbuf], add=True)

        @plsc.parallel_loop(0, _CH, 1)
        def _gc(i):
            g = lax.shift_right_logical(i, 4)
            j = i & 15
            dvec = dstbuf[pl.ds(g * 16, 16)]
            dstb = _vgather(dvec, jnp.full((16,), j, jnp.int32))
            rvec = (dstb & 1) * 8
            slot = (lax.shift_right_logical(dstb[0], 1) & 7) * 16
            exv = exbuf[pl.ds(i * 16, 16)]
            idxs = (lane - rvec) & 15
            denv = jnp.where((lane >= rvec) & (lane < rvec + 8),
                             _vgather(exv, idxs), 0.0)
            for sj in range(8):
                msgbuf[i, pl.ds(sj * 16, 16)] = zero16
            msgbuf[i, pl.ds(slot, 16)] = denv
        pltpu.sync_copy(msgbuf, den_sp.at[srcbuf], add=True)
        return carry
    lax.fori_loop(0, _NCH, _chunk, 0)

    plsc.subcore_barrier()
    pltpu.sync_copy(aug.at[pl.ds(nbase, _NPT)],
                    acc_hbm.at[c, pl.ds(nbase, _NPT)])
    nd = _NPAD // 16 // 16
    pltpu.sync_copy(den_sp.at[pl.ds(s * nd, nd)],
                    den_hbm.at[c, pl.ds(s * nd, nd)])


def _edge_pass(q, kv, ebias):
    mesh = plsc.VectorSubcoreMesh(core_axis_name="c", subcore_axis_name="s")
    f = pl.kernel(
        _edge_body,
        out_type=[
            jax.ShapeDtypeStruct((2, _NPAD, _D), jnp.float32),
            jax.ShapeDtypeStruct((2, _NPAD // 16, _D), jnp.float32),
        ],
        mesh=mesh,
        scratch_types=[
            pltpu.VMEM((_CH, 2 * _D), jnp.float32),
            pltpu.VMEM((_CH, _D), jnp.float32),
            pltpu.VMEM((_CH, _D), jnp.float32),
            pltpu.VMEM((_CH,), jnp.int32),
            pltpu.VMEM((_CH,), jnp.int32),
            pltpu.VMEM((_CH * 16,), jnp.float32),
            pltpu.VMEM((_CH * 16,), jnp.float32),
            pltpu.VMEM_SHARED((_NPAD, _D), jnp.float32),
            pltpu.VMEM_SHARED((_NPAD // 16, _D), jnp.float32),
            pltpu.SemaphoreType.DMA,
            pltpu.SemaphoreType.DMA,
        ],
        compiler_params=pltpu.CompilerParams(needs_layout_passes=False),
    )
    return f(q, kv, ebias)


def _comb_body(acc_ref, den_ref, x_ref, wo_ref, bo_ref, g2_ref, b2_ref,
               w1_ref, bf1_ref, w2_ref, bf2_ref, y_ref):
    acc = acc_ref[0] + acc_ref[1]
    den = den_ref[0] + den_ref[1]
    li = lax.broadcasted_iota(jnp.int32, (_H, _D), 1) // _DH
    hi = lax.broadcasted_iota(jnp.int32, (_H, _D), 0)
    rep = (li == hi).astype(jnp.float32)
    dn = (((1,), (0,)), ((), ()))
    den_r = lax.dot_general(den, rep, dn, preferred_element_type=jnp.float32)
    out = acc / (den_r + 1e-16)
    dt = (((1,), (1,)), ((), ()))
    attn = lax.dot_general(out, wo_ref[...], dt,
                           preferred_element_type=jnp.float32) + bo_ref[...]
    h = x_ref[...] + attn
    hn2 = _ln(h, g2_ref[...], b2_ref[...])
    f1 = lax.dot_general(hn2, w1_ref[...], dt,
                         preferred_element_type=jnp.float32) + bf1_ref[...]
    f1 = jnp.maximum(f1, 0.0)
    ffn = lax.dot_general(f1, w2_ref[...], dt,
                          preferred_element_type=jnp.float32) + bf2_ref[...]
    y_ref[...] = h + ffn


def _combine(acc, den, x, Wo, bo, g2, b2, W1, bf1, W2, bf2):
    blk = 400
    grid = _N // blk
    full = lambda s: pl.BlockSpec(s, lambda i: tuple(0 for _ in s))
    return pl.pallas_call(
        _comb_body,
        grid=(grid,),
        in_specs=[
            pl.BlockSpec((2, blk, _D), lambda i: (0, i, 0)),
            pl.BlockSpec((2, blk, _H), lambda i: (0, i, 0)),
            pl.BlockSpec((blk, _D), lambda i: (i, 0)),
            full((_D, _D)), full((1, _D)),
            full((1, _D)), full((1, _D)),
            full((2 * _D, _D)), full((1, 2 * _D)),
            full((_D, 2 * _D)), full((1, _D)),
        ],
        out_specs=pl.BlockSpec((blk, _D), lambda i: (i, 0)),
        out_shape=jax.ShapeDtypeStruct((_N, _D), jnp.float32),
    )(acc, den, x, Wo, bo.reshape(1, _D), g2.reshape(1, _D), b2.reshape(1, _D),
      W1, bf1.reshape(1, 2 * _D), W2, bf2.reshape(1, _D))


def kernel(x, edge_index, edge_attr, Wq, bq, Wk, bk, Wv, bv, We, be,
           Wo, bo, g1, b1, g2, b2, W1, bf1, W2, bf2):
    src = edge_index[0]
    dst = edge_index[1]
    q, kv = _qkv(x, Wq, bq, Wk, bk, Wv, bv, g1, b1)
    ebias = _ebias(edge_attr, We, be, src, dst).reshape(-1)
    acc, den = _edge_pass(q, kv, ebias)
    den = den.reshape(2, _NPAD, _H)
    return _combine(acc, den, x, Wo, bo, g2, b2, W1, bf1, W2, bf2)


# speedup vs baseline: 2.6408x; 1.0606x over previous
import functools

import jax
import jax.numpy as jnp
from jax import lax
from jax.experimental import pallas as pl
from jax.experimental.pallas import tpu as pltpu
from jax.experimental.pallas import tpu_sc as plsc

_GDN = lax.GatherDimensionNumbers(
    offset_dims=(), collapsed_slice_dims=(0,), start_index_map=(0,))


def _vgather(v, idx):
    return lax.gather(v, idx[:, None], _GDN, (1,),
                      mode=lax.GatherScatterMode.PROMISE_IN_BOUNDS)


_N = 10000
_E = 320000
_D = 128
_H = 8
_DH = 16

_NTILES = 32
_EPT = _E // _NTILES
_CH = 80
_NCH = _EPT // _CH
_GR = _CH // 16
_NPAD = 10240
_NPT = _NPAD // 16
_AW = 144


def _ln(x, g, b):
    mu = jnp.mean(x, axis=-1, keepdims=True)
    var = jnp.mean((x - mu) * (x - mu), axis=-1, keepdims=True)
    return (x - mu) / jnp.sqrt(var + 1e-5) * g + b


def _qkv_body(x_ref, wq_ref, bq_ref, wk_ref, bk_ref, wv_ref, bv_ref,
              g1_ref, b1_ref, q_ref, kv_ref):
    hn = _ln(x_ref[...], g1_ref[...], b1_ref[...])
    dn = (((1,), (1,)), ((), ()))
    q = lax.dot_general(hn, wq_ref[...], dn,
                        preferred_element_type=jnp.float32) + bq_ref[...]
    k = lax.dot_general(hn, wk_ref[...], dn,
                        preferred_element_type=jnp.float32) + bk_ref[...]
    v = lax.dot_general(hn, wv_ref[...], dn,
                        preferred_element_type=jnp.float32) + bv_ref[...]
    q_ref[...] = q * 0.25
    kv_ref[...] = jnp.concatenate([k, v], axis=1)


def _qkv(x, Wq, bq, Wk, bk, Wv, bv, g1, b1):
    blk = 400
    grid = _N // blk
    full = lambda s: pl.BlockSpec(s, lambda i: (0, 0))
    return pl.pallas_call(
        _qkv_body,
        grid=(grid,),
        in_specs=[
            pl.BlockSpec((blk, _D), lambda i: (i, 0)),
            full((_D, _D)), full((1, _D)),
            full((_D, _D)), full((1, _D)),
            full((_D, _D)), full((1, _D)),
            full((1, _D)), full((1, _D)),
        ],
        out_specs=[
            pl.BlockSpec((blk, _D), lambda i: (i, 0)),
            pl.BlockSpec((blk, 2 * _D), lambda i: (i, 0)),
        ],
        out_shape=[
            jax.ShapeDtypeStruct((_N, _D), jnp.float32),
            jax.ShapeDtypeStruct((_N, 2 * _D), jnp.float32),
        ],
    )(x, Wq, bq.reshape(1, _D), Wk, bk.reshape(1, _D),
      Wv, bv.reshape(1, _D), g1.reshape(1, _D), b1.reshape(1, _D))


def _ebias_body(ea_ref, we_ref, be_ref, src_ref, dst_ref, out_ref):
    dn = (((1,), (1,)), ((), ()))
    eb = lax.dot_general(ea_ref[...], we_ref[...], dn,
                         preferred_element_type=jnp.float32) + be_ref[...]
    srcf = lax.bitcast_convert_type(src_ref[...], jnp.float32)
    dstf = lax.bitcast_convert_type(dst_ref[...], jnp.float32)
    blk = eb.shape[0]
    out_ref[...] = jnp.concatenate(
        [eb, srcf, dstf, jnp.zeros((blk, 6), jnp.float32)], axis=1)


def _ebias(edge_attr, We, be, src, dst):
    blk = 2000
    grid = _E // blk
    return pl.pallas_call(
        _ebias_body,
        grid=(grid,),
        in_specs=[
            pl.BlockSpec((blk, _D), lambda i: (i, 0)),
            pl.BlockSpec((_H, _D), lambda i: (0, 0)),
            pl.BlockSpec((1, _H), lambda i: (0, 0)),
            pl.BlockSpec((blk, 1), lambda i: (i, 0)),
            pl.BlockSpec((blk, 1), lambda i: (i, 0)),
        ],
        out_specs=pl.BlockSpec((blk, 2 * _H), lambda i: (i, 0)),
        out_shape=jax.ShapeDtypeStruct((_E, 2 * _H), jnp.float32),
    )(edge_attr, We, be.reshape(1, _H), src.reshape(_E, 1), dst.reshape(_E, 1))


def _edge_body(q_hbm, kv_hbm, eb_hbm,
               acc_hbm, den_hbm,
               kvbuf, qbuf, msgbuf, srcbuf, dstbuf, ebuf, exbuf, aug, den_sp,
               sem1, sem2, sem3):
    c = lax.axis_index("c")
    s = lax.axis_index("s")
    wid = c * 16 + s
    ebase = wid * _EPT
    nbase = s * _NPT
    zero16 = jnp.zeros((16,), jnp.float32)
    lane = lax.iota(jnp.int32, 16)

    def _zrow(i, carry):
        for j in range(_D // 16):
            msgbuf[i, pl.ds(j * 16, 16)] = zero16
        return carry
    lax.fori_loop(0, _CH, _zrow, 0)
    for j in range(_NPT // _CH):
        pltpu.sync_copy(msgbuf, aug.at[pl.ds(nbase + j * _CH, _CH)])
    pltpu.sync_copy(msgbuf.at[pl.ds(0, _NPAD // 16 // 16)],
                    den_sp.at[pl.ds(s * (_NPAD // 16 // 16), _NPAD // 16 // 16)])
    plsc.subcore_barrier()
    pltpu.async_copy(eb_hbm.at[pl.ds(ebase * 16, _CH * 16)], ebuf, sem3)

    def _chunk(ci, carry):
        base = ebase + ci * _CH
        pltpu.make_async_copy(eb_hbm.at[pl.ds(base * 16, _CH * 16)],
                              ebuf, sem3).wait()

        @plsc.parallel_loop(0, _GR, 1)
        def _unpack(g):
            e_idx = g * 16 + lane
            sv = plsc.bitcast(plsc.load_gather(ebuf, [e_idx * 16 + _H]),
                              jnp.int32)
            dv = plsc.bitcast(plsc.load_gather(ebuf, [e_idx * 16 + _H + 1]),
                              jnp.int32)
            srcbuf[pl.ds(g * 16, 16)] = sv
            dstbuf[pl.ds(g * 16, 16)] = dv

        kvd = pltpu.async_copy(kv_hbm.at[srcbuf], kvbuf, sem1)
        qd = pltpu.async_copy(q_hbm.at[dstbuf], qbuf, sem2)
        kvd.wait()
        qd.wait()

        @plsc.parallel_loop(0, _GR, 1)
        def _gsrc(g):
            dv = dstbuf[pl.ds(g * 16, 16)]
            srcbuf[pl.ds(g * 16, 16)] = lax.shift_right_logical(dv, 4)

        @plsc.parallel_loop(0, _CH, 1)
        def _ga(i):
            eb_row = ebuf[pl.ds(i * 16, 16)]
            sums = jnp.zeros((16,), jnp.float32)
            for h in range(_H):
                kvh = kvbuf[i, pl.ds(h * _DH, _DH)]
                qh = qbuf[i, pl.ds(h * _DH, _DH)]
                cs = plsc.cumsum(kvh * qh)
                sums = sums + jnp.where(lane == h,
                                        jnp.full((16,), cs[15], jnp.float32),
                                        0.0)
            exv = jnp.exp(sums + eb_row)
            for h in range(_H):
                exb = jnp.full((16,), exv[h], jnp.float32)
                vh = kvbuf[i, pl.ds(_D + h * _DH, _DH)]
                msgbuf[i, pl.ds(h * _DH, _DH)] = vh * exb
            exbuf[pl.ds(i * 16, 16)] = exv
        nb = jnp.minimum(base + _CH, _E - _CH) * 16
        pltpu.async_copy(eb_hbm.at[pl.ds(nb, _CH * 16)], ebuf, sem3)
        pltpu.sync_copy(msgbuf, aug.at[dstbuf], add=True)

        @plsc.parallel_loop(0, _CH, 1)
        def _gc(i):
            g = lax.shift_right_logical(i, 4)
            j = i & 15
            dvec = dstbuf[pl.ds(g * 16, 16)]
            dstb = _vgather(dvec, jnp.full((16,), j, jnp.int32))
            rvec = (dstb & 1) * 8
            slot = (lax.shift_right_logical(dstb[0], 1) & 7) * 16
            exv = exbuf[pl.ds(i * 16, 16)]
            idxs = (lane - rvec) & 15
            denv = jnp.where((lane >= rvec) & (lane < rvec + 8),
                             _vgather(exv, idxs), 0.0)
            for sj in range(8):
                msgbuf[i, pl.ds(sj * 16, 16)] = zero16
            msgbuf[i, pl.ds(slot, 16)] = denv
        pltpu.sync_copy(msgbuf, den_sp.at[srcbuf], add=True)
        return carry
    lax.fori_loop(0, _NCH, _chunk, 0)
    pltpu.make_async_copy(eb_hbm.at[pl.ds(0, _CH * 16)], ebuf, sem3).wait()

    plsc.subcore_barrier()
    pltpu.sync_copy(aug.at[pl.ds(nbase, _NPT)],
                    acc_hbm.at[c, pl.ds(nbase, _NPT)])
    nd = _NPAD // 16 // 16
    pltpu.sync_copy(den_sp.at[pl.ds(s * nd, nd)],
                    den_hbm.at[c, pl.ds(s * nd, nd)])


def _edge_pass(q, kv, ebias):
    mesh = plsc.VectorSubcoreMesh(core_axis_name="c", subcore_axis_name="s")
    f = pl.kernel(
        _edge_body,
        out_type=[
            jax.ShapeDtypeStruct((2, _NPAD, _D), jnp.float32),
            jax.ShapeDtypeStruct((2, _NPAD // 16, _D), jnp.float32),
        ],
        mesh=mesh,
        scratch_types=[
            pltpu.VMEM((_CH, 2 * _D), jnp.float32),
            pltpu.VMEM((_CH, _D), jnp.float32),
            pltpu.VMEM((_CH, _D), jnp.float32),
            pltpu.VMEM((_CH,), jnp.int32),
            pltpu.VMEM((_CH,), jnp.int32),
            pltpu.VMEM((_CH * 16,), jnp.float32),
            pltpu.VMEM((_CH * 16,), jnp.float32),
            pltpu.VMEM_SHARED((_NPAD, _D), jnp.float32),
            pltpu.VMEM_SHARED((_NPAD // 16, _D), jnp.float32),
            pltpu.SemaphoreType.DMA,
            pltpu.SemaphoreType.DMA,
            pltpu.SemaphoreType.DMA,
        ],
        compiler_params=pltpu.CompilerParams(needs_layout_passes=False),
    )
    return f(q, kv, ebias)


def _comb_body(acc_ref, den_ref, x_ref, wo_ref, bo_ref, g2_ref, b2_ref,
               w1_ref, bf1_ref, w2_ref, bf2_ref, y_ref):
    acc = acc_ref[0] + acc_ref[1]
    den = den_ref[0] + den_ref[1]
    li = lax.broadcasted_iota(jnp.int32, (_H, _D), 1) // _DH
    hi = lax.broadcasted_iota(jnp.int32, (_H, _D), 0)
    rep = (li == hi).astype(jnp.float32)
    dn = (((1,), (0,)), ((), ()))
    den_r = lax.dot_general(den, rep, dn, preferred_element_type=jnp.float32)
    out = acc / (den_r + 1e-16)
    dt = (((1,), (1,)), ((), ()))
    attn = lax.dot_general(out, wo_ref[...], dt,
                           preferred_element_type=jnp.float32) + bo_ref[...]
    h = x_ref[...] + attn
    hn2 = _ln(h, g2_ref[...], b2_ref[...])
    f1 = lax.dot_general(hn2, w1_ref[...], dt,
                         preferred_element_type=jnp.float32) + bf1_ref[...]
    f1 = jnp.maximum(f1, 0.0)
    ffn = lax.dot_general(f1, w2_ref[...], dt,
                          preferred_element_type=jnp.float32) + bf2_ref[...]
    y_ref[...] = h + ffn


def _combine(acc, den, x, Wo, bo, g2, b2, W1, bf1, W2, bf2):
    blk = 400
    grid = _N // blk
    full = lambda s: pl.BlockSpec(s, lambda i: tuple(0 for _ in s))
    return pl.pallas_call(
        _comb_body,
        grid=(grid,),
        in_specs=[
            pl.BlockSpec((2, blk, _D), lambda i: (0, i, 0)),
            pl.BlockSpec((2, blk, _H), lambda i: (0, i, 0)),
            pl.BlockSpec((blk, _D), lambda i: (i, 0)),
            full((_D, _D)), full((1, _D)),
            full((1, _D)), full((1, _D)),
            full((2 * _D, _D)), full((1, 2 * _D)),
            full((_D, 2 * _D)), full((1, _D)),
        ],
        out_specs=pl.BlockSpec((blk, _D), lambda i: (i, 0)),
        out_shape=jax.ShapeDtypeStruct((_N, _D), jnp.float32),
    )(acc, den, x, Wo, bo.reshape(1, _D), g2.reshape(1, _D), b2.reshape(1, _D),
      W1, bf1.reshape(1, 2 * _D), W2, bf2.reshape(1, _D))


def kernel(x, edge_index, edge_attr, Wq, bq, Wk, bk, Wv, bv, We, be,
           Wo, bo, g1, b1, g2, b2, W1, bf1, W2, bf2):
    src = edge_index[0]
    dst = edge_index[1]
    q, kv = _qkv(x, Wq, bq, Wk, bk, Wv, bv, g1, b1)
    ebias = _ebias(edge_attr, We, be, src, dst).reshape(-1)
    acc, den = _edge_pass(q, kv, ebias)
    den = den.reshape(2, _NPAD, _H)
    return _combine(acc, den, x, Wo, bo, g2, b2, W1, bf1, W2, bf2)
